# baseline (speedup 1.0000x reference)
import jax
import jax.numpy as jnp
from jax import lax
from jax.experimental import pallas as pl
from jax.experimental.pallas import tpu as pltpu
from jax.experimental.pallas import tpu_sc as plsc

_NUM_CORES = 2
_NW = 32
_CHUNK = 32
_NBUF = 3
_LA = 2
_SC_ROWS = 2048
_TC_BLOCK = 2048


def _sc_body(w_hbm, out_hbm, buf, g0, g1, g2, s0, s1, s2):
    gsems = (g0, g1, g2)
    ssems = (s0, s1, s2)
    rows = out_hbm.shape[0]
    rpw = rows // _NW
    nchunks = rpw // _CHUNK
    wid = lax.axis_index("s") * _NUM_CORES + lax.axis_index("c")
    base = wid * rpw

    def gather(j):
        k = j % _NBUF
        return pltpu.async_copy(
            w_hbm.at[pl.ds(base + j * _CHUNK, _CHUNK)], buf.at[k], gsems[k]
        )

    def scatter(i):
        k = i % _NBUF
        return pltpu.async_copy(
            buf.at[k], out_hbm.at[pl.ds(base + i * _CHUNK, _CHUNK)], ssems[k]
        )

    gd, sd = {}, {}
    for j in range(_LA):
        gd[j] = gather(j)
    for i in range(nchunks):
        gd[i].wait()
        sd[i] = scatter(i)
        j = i + _LA
        if j < nchunks:
            if j >= _NBUF:
                sd[j - _NBUF].wait()
            gd[j] = gather(j)
    for i in range(max(0, nchunks - _NBUF), nchunks):
        sd[i].wait()


def _tc_body(w_ref, out_ref):
    out_ref[...] = w_ref[...]


def kernel(x, W):
    seq_len = x.shape[1]
    dim = W.shape[1]
    tc_rows = seq_len - _SC_ROWS

    mesh = plsc.VectorSubcoreMesh(core_axis_name="c", subcore_axis_name="s")
    sc_k = pl.kernel(
        _sc_body,
        out_type=jax.ShapeDtypeStruct((_SC_ROWS, dim), W.dtype),
        mesh=mesh,
        scratch_types=[pltpu.VMEM((_NBUF, _CHUNK, dim), jnp.float32)]
        + [pltpu.SemaphoreType.DMA] * 6,
    )
    tail = sc_k(lax.slice_in_dim(W, tc_rows, seq_len, axis=0))

    head = pl.pallas_call(
        _tc_body,
        out_shape=jax.ShapeDtypeStruct((tc_rows, dim), W.dtype),
        grid=(tc_rows // _TC_BLOCK,),
        in_specs=[pl.BlockSpec((_TC_BLOCK, dim), lambda i: (i, 0))],
        out_specs=pl.BlockSpec((_TC_BLOCK, dim), lambda i: (i, 0)),
    )(W)

    return jnp.concatenate([head, tail], axis=0)


# speedup vs baseline: 1.5831x; 1.5831x over previous
import jax
import jax.numpy as jnp
from jax import lax
from jax.experimental import pallas as pl
from jax.experimental.pallas import tpu as pltpu
from jax.experimental.pallas import tpu_sc as plsc

_NUM_CORES = 2
_NW = 32

_TEC_ROWS = 4096
_TC_CHUNK = 32
_TC_NBUF = 3
_TC_LA = 2
_SC_CHUNK = 128
_SC_NBUF = 3
_SC_LA = 2


def _ring_copy(w_hbm, out_hbm, buf, gsems, ssems, base, total_rows, chunk,
               nbuf, la):
    nchunks = total_rows // chunk

    def gather(j):
        k = j % nbuf
        return pltpu.async_copy(
            w_hbm.at[pl.ds(base + j * chunk, chunk)], buf.at[k], gsems[k]
        )

    def scatter(i):
        k = i % nbuf
        return pltpu.async_copy(
            buf.at[k], out_hbm.at[pl.ds(base + i * chunk, chunk)], ssems[k]
        )

    gd, sd = {}, {}
    for j in range(min(la, nchunks)):
        gd[j] = gather(j)
    for i in range(nchunks):
        gd[i].wait()
        sd[i] = scatter(i)
        j = i + la
        if j < nchunks:
            if j >= nbuf:
                sd[j - nbuf].wait()
            gd[j] = gather(j)
    for i in range(max(0, nchunks - nbuf), nchunks):
        sd[i].wait()


def _tec_fn(w_hbm, out_hbm, tbuf, sbuf, *sems):
    gsems = sems[:_TC_NBUF]
    ssems = sems[_TC_NBUF:2 * _TC_NBUF]
    rpw = _TEC_ROWS // _NW
    wid = lax.axis_index("s") * _NUM_CORES + lax.axis_index("c")
    _ring_copy(w_hbm, out_hbm, tbuf, gsems, ssems, wid * rpw, rpw,
               _TC_CHUNK, _TC_NBUF, _TC_LA)


def _scs_fn(w_hbm, out_hbm, tbuf, sbuf, *sems):
    gsems = sems[2 * _TC_NBUF:2 * _TC_NBUF + _SC_NBUF]
    ssems = sems[2 * _TC_NBUF + _SC_NBUF:]
    rows = out_hbm.shape[0]
    rpc = (rows - _TEC_ROWS) // _NUM_CORES
    cid = lax.axis_index("c")
    _ring_copy(w_hbm, out_hbm, sbuf, gsems, ssems, _TEC_ROWS + cid * rpc,
               rpc, _SC_CHUNK, _SC_NBUF, _SC_LA)


def kernel(x, W):
    seq_len = x.shape[1]
    dim = W.shape[1]
    tec_mesh = plsc.VectorSubcoreMesh(core_axis_name="c", subcore_axis_name="s")
    scs_mesh = plsc.ScalarSubcoreMesh(axis_name="c")
    k = pl.kernel(
        [_tec_fn, _scs_fn],
        out_type=jax.ShapeDtypeStruct((seq_len, dim), W.dtype),
        mesh=[tec_mesh, scs_mesh],
        scratch_types=(
            [(pltpu.VMEM @ tec_mesh)((_TC_NBUF, _TC_CHUNK, dim), jnp.float32),
             pltpu.VMEM_SHARED((_SC_NBUF, _SC_CHUNK, dim), jnp.float32)]
            + [pltpu.SemaphoreType.DMA @ tec_mesh] * (2 * _TC_NBUF)
            + [pltpu.SemaphoreType.DMA @ scs_mesh] * (2 * _SC_NBUF)
        ),
    )
    return k(W)


# speedup vs baseline: 1.6580x; 1.0473x over previous
import jax
import jax.numpy as jnp
from jax import lax
from jax.experimental import pallas as pl
from jax.experimental.pallas import tpu as pltpu
from jax.experimental.pallas import tpu_sc as plsc

_NUM_CORES = 2
_NW = 32

_TEC_ROWS = 5120
_TC_CHUNK = 16
_TC_NBUF = 4
_TC_LA = 2
_SC_CHUNK = 256
_SC_NBUF = 3
_SC_LA = 2


def _ring_copy(w_hbm, out_hbm, buf, gsems, ssems, base, total_rows, chunk,
               nbuf, la):
    nchunks = total_rows // chunk

    def gather(j):
        k = j % nbuf
        return pltpu.async_copy(
            w_hbm.at[pl.ds(base + j * chunk, chunk)], buf.at[k], gsems[k]
        )

    def scatter(i):
        k = i % nbuf
        return pltpu.async_copy(
            buf.at[k], out_hbm.at[pl.ds(base + i * chunk, chunk)], ssems[k]
        )

    gd, sd = {}, {}
    for j in range(min(la, nchunks)):
        gd[j] = gather(j)
    for i in range(nchunks):
        gd[i].wait()
        sd[i] = scatter(i)
        j = i + la
        if j < nchunks:
            if j >= nbuf:
                sd[j - nbuf].wait()
            gd[j] = gather(j)
    for i in range(max(0, nchunks - nbuf), nchunks):
        sd[i].wait()


def _tec_fn(w_hbm, out_hbm, tbuf, sbuf, *sems):
    gsems = sems[:_TC_NBUF]
    ssems = sems[_TC_NBUF:2 * _TC_NBUF]
    rpw = _TEC_ROWS // _NW
    wid = lax.axis_index("s") * _NUM_CORES + lax.axis_index("c")
    _ring_copy(w_hbm, out_hbm, tbuf, gsems, ssems, wid * rpw, rpw,
               _TC_CHUNK, _TC_NBUF, _TC_LA)


def _scs_fn(w_hbm, out_hbm, tbuf, sbuf, *sems):
    gsems = sems[2 * _TC_NBUF:2 * _TC_NBUF + _SC_NBUF]
    ssems = sems[2 * _TC_NBUF + _SC_NBUF:]
    rows = out_hbm.shape[0]
    rpc = (rows - _TEC_ROWS) // _NUM_CORES
    cid = lax.axis_index("c")
    _ring_copy(w_hbm, out_hbm, sbuf, gsems, ssems, _TEC_ROWS + cid * rpc,
               rpc, _SC_CHUNK, _SC_NBUF, _SC_LA)


def kernel(x, W):
    seq_len = x.shape[1]
    dim = W.shape[1]
    tec_mesh = plsc.VectorSubcoreMesh(core_axis_name="c", subcore_axis_name="s")
    scs_mesh = plsc.ScalarSubcoreMesh(axis_name="c")
    k = pl.kernel(
        [_tec_fn, _scs_fn],
        out_type=jax.ShapeDtypeStruct((seq_len, dim), W.dtype),
        mesh=[tec_mesh, scs_mesh],
        scratch_types=(
            [(pltpu.VMEM @ tec_mesh)((_TC_NBUF, _TC_CHUNK, dim), jnp.float32),
             pltpu.VMEM_SHARED((_SC_NBUF, _SC_CHUNK, dim), jnp.float32)]
            + [pltpu.SemaphoreType.DMA @ tec_mesh] * (2 * _TC_NBUF)
            + [pltpu.SemaphoreType.DMA @ scs_mesh] * (2 * _SC_NBUF)
        ),
    )
    return k(W)
